# baseline (speedup 1.0000x reference)
import jax
import jax.numpy as jnp
from jax.experimental import pallas as pl
from jax.experimental.pallas import tpu as pltpu

A = 21
PSEUDOCOUNT = 0.001
EPS = 1e-9
N_SEQS = 512
L = 256
PB = 64
NB = L // PB
TIA = PB * A
INV_LN2 = 1.4426950408889634
LOG2A = 4.392317422778761


def _mi_tiles_kernel(xi_ref, xj_ref, s1_ref, cnt_ref):
    xi = xi_ref[0]
    xj = xj_ref[0]
    counts = jax.lax.dot_general(
        xi, xj, (((0,), (0,)), ((), ())), preferred_element_type=jnp.float32)
    y = counts * (1.0 / N_SEQS) + EPS
    t = y * (jnp.log(y) * INV_LN2)
    th = t.astype(jnp.bfloat16)
    tl = (t - th.astype(jnp.float32)).astype(jnp.bfloat16)
    g_r = jax.lax.broadcasted_iota(jnp.int32, (TIA, PB), 0) // A
    g_c = jax.lax.broadcasted_iota(jnp.int32, (TIA, PB), 1)
    g = (g_r == g_c).astype(jnp.bfloat16)
    tmp = (jnp.dot(th, g, preferred_element_type=jnp.float32)
           + jnp.dot(tl, g, preferred_element_type=jnp.float32))
    g2_r = jax.lax.broadcasted_iota(jnp.int32, (PB, TIA), 0)
    g2_c = jax.lax.broadcasted_iota(jnp.int32, (PB, TIA), 1) // A
    g2 = (g2_r == g2_c).astype(jnp.bfloat16)
    tmp_h = tmp.astype(jnp.bfloat16)
    tmp_l = (tmp - tmp_h.astype(jnp.float32)).astype(jnp.bfloat16)
    s1_ref[0, 0] = (jnp.dot(g2, tmp_h, preferred_element_type=jnp.float32)
                    + jnp.dot(g2, tmp_l, preferred_element_type=jnp.float32))
    cnt_ref[...] = jnp.sum(xi.astype(jnp.float32), axis=0).reshape(1, 1, TIA)


def _finalize_kernel(cnt_ref, s1_ref, pssm_ref, cons_ref, coev_ref):
    cnt = cnt_ref[...]
    mean = cnt * (1.0 / N_SEQS)
    freq = mean + PSEUDOCOUNT
    fn = freq / jnp.sum(freq, axis=1, keepdims=True)
    pssm_ref[...] = jnp.log(fn * float(A)) * INV_LN2
    fe = mean + EPS
    neg_ent = jnp.sum(fe * (jnp.log(fe) * INV_LN2), axis=1, keepdims=True)
    cons_ref[...] = 1.0 + neg_ent * (1.0 / LOG2A)
    logp = jnp.log(fe) * INV_LN2
    c = jnp.sum((mean + A * EPS) * logp, axis=1, keepdims=True)
    mi = s1_ref[...] - c - jnp.transpose(c)
    ii = jax.lax.broadcasted_iota(jnp.int32, (L, L), 0)
    jj = jax.lax.broadcasted_iota(jnp.int32, (L, L), 1)
    mi = jnp.where(ii == jj, 0.0, mi)
    row_mean = jnp.mean(mi, axis=1, keepdims=True)
    col_mean = jnp.mean(mi, axis=0, keepdims=True)
    total = jnp.mean(mi)
    coev_ref[...] = mi - row_mean * col_mean / (total + EPS)


@jax.jit
def kernel(msa):
    mb = msa.astype(jnp.bfloat16).reshape(N_SEQS, NB, TIA).transpose(1, 0, 2)
    s1, cnt = pl.pallas_call(
        _mi_tiles_kernel,
        grid=(NB, NB),
        in_specs=[
            pl.BlockSpec((1, N_SEQS, TIA), lambda i, j: (i, 0, 0)),
            pl.BlockSpec((1, N_SEQS, TIA), lambda i, j: (j, 0, 0)),
        ],
        out_specs=[
            pl.BlockSpec((1, 1, PB, PB), lambda i, j: (i, j, 0, 0)),
            pl.BlockSpec((1, 1, TIA), lambda i, j: (i, 0, 0)),
        ],
        out_shape=[
            jax.ShapeDtypeStruct((NB, NB, PB, PB), jnp.float32),
            jax.ShapeDtypeStruct((NB, 1, TIA), jnp.float32),
        ],
        compiler_params=pltpu.CompilerParams(
            dimension_semantics=("parallel", "arbitrary"),
            vmem_limit_bytes=56 * 1024 * 1024,
        ),
        name="mi_tiles",
    )(mb, mb)
    s1 = s1.transpose(0, 2, 1, 3).reshape(L, L)
    cnt = cnt.reshape(L, A)
    pssm, cons, coev = pl.pallas_call(
        _finalize_kernel,
        out_shape=[
            jax.ShapeDtypeStruct((L, A), jnp.float32),
            jax.ShapeDtypeStruct((L, 1), jnp.float32),
            jax.ShapeDtypeStruct((L, L), jnp.float32),
        ],
        name="finalize",
    )(cnt, s1)
    return pssm, cons.reshape(L), coev


# speedup vs baseline: 1.5050x; 1.5050x over previous
import jax
import jax.numpy as jnp
from jax.experimental import pallas as pl
from jax.experimental.pallas import tpu as pltpu

A = 21
PSEUDOCOUNT = 0.001
EPS = 1e-9
N_SEQS = 512
L = 256
PB = 64
NB = L // PB
TIA = PB * A
INV_LN2 = 1.4426950408889634
LOG2A = 4.392317422778761


def _mi_tiles_kernel(xi_ref, xj_ref, s1_ref, cnt_ref):
    xi = xi_ref[0]
    xj = xj_ref[0]
    counts = jax.lax.dot_general(
        xi, xj, (((0,), (0,)), ((), ())), preferred_element_type=jnp.float32)
    y = counts * (1.0 / N_SEQS) + EPS
    t = y * (jnp.log(y) * INV_LN2)
    acc = t[:, 0:128]
    for c in range(1, 10):
        acc = acc + t[:, 128 * c:128 * (c + 1)]
    part = acc[:, 0:PB] + acc[:, PB:128] + t[:, 20 * PB:TIA]
    s = part[0:PB, :]
    for a in range(1, A):
        s = s + part[PB * a:PB * (a + 1), :]
    s1_ref[0, 0] = s
    @pl.when(pl.program_id(1) == 0)
    def _():
        cnt_ref[...] = jnp.sum(xi.astype(jnp.float32), axis=0).reshape(1, 1, TIA)


def _finalize_kernel(cnt_ref, s1_ref, pssm_ref, cons_ref, coev_ref):
    cnt = cnt_ref[...]
    mean = cnt * (1.0 / N_SEQS)
    freq = mean + PSEUDOCOUNT
    fn = freq / jnp.sum(freq, axis=1, keepdims=True)
    pssm_ref[...] = jnp.log(fn * float(A)) * INV_LN2
    fe = mean + EPS
    neg_ent = jnp.sum(fe * (jnp.log(fe) * INV_LN2), axis=1, keepdims=True)
    cons_ref[...] = 1.0 + neg_ent * (1.0 / LOG2A)
    logp = jnp.log(fe) * INV_LN2
    c = jnp.sum((mean + A * EPS) * logp, axis=1, keepdims=True)
    mi = s1_ref[...] - c - jnp.transpose(c)
    ii = jax.lax.broadcasted_iota(jnp.int32, (L, L), 0)
    jj = jax.lax.broadcasted_iota(jnp.int32, (L, L), 1)
    mi = jnp.where(ii == jj, 0.0, mi)
    row_mean = jnp.mean(mi, axis=1, keepdims=True)
    col_mean = jnp.mean(mi, axis=0, keepdims=True)
    total = jnp.mean(mi)
    coev_ref[...] = mi - row_mean * col_mean / (total + EPS)


@jax.jit
def kernel(msa):
    mb = (msa.astype(jnp.bfloat16)
          .reshape(N_SEQS, NB, PB, A)
          .transpose(1, 0, 3, 2)
          .reshape(NB, N_SEQS, TIA))
    s1, cnt = pl.pallas_call(
        _mi_tiles_kernel,
        grid=(NB, NB),
        in_specs=[
            pl.BlockSpec((1, N_SEQS, TIA), lambda i, j: (i, 0, 0)),
            pl.BlockSpec((1, N_SEQS, TIA), lambda i, j: (j, 0, 0)),
        ],
        out_specs=[
            pl.BlockSpec((1, 1, PB, PB), lambda i, j: (i, j, 0, 0)),
            pl.BlockSpec((1, 1, TIA), lambda i, j: (i, 0, 0)),
        ],
        out_shape=[
            jax.ShapeDtypeStruct((NB, NB, PB, PB), jnp.float32),
            jax.ShapeDtypeStruct((NB, 1, TIA), jnp.float32),
        ],
        compiler_params=pltpu.CompilerParams(
            dimension_semantics=("parallel", "arbitrary"),
            vmem_limit_bytes=56 * 1024 * 1024,
        ),
        name="mi_tiles",
    )(mb, mb)
    s1 = s1.transpose(0, 2, 1, 3).reshape(L, L)
    cnt = cnt.reshape(NB, A, PB).transpose(0, 2, 1).reshape(L, A)
    pssm, cons, coev = pl.pallas_call(
        _finalize_kernel,
        out_shape=[
            jax.ShapeDtypeStruct((L, A), jnp.float32),
            jax.ShapeDtypeStruct((L, 1), jnp.float32),
            jax.ShapeDtypeStruct((L, L), jnp.float32),
        ],
        name="finalize",
    )(cnt, s1)
    return pssm, cons.reshape(L), coev


# speedup vs baseline: 1.9509x; 1.2962x over previous
import jax
import jax.numpy as jnp
from jax.experimental import pallas as pl
from jax.experimental.pallas import tpu as pltpu

A = 21
PSEUDOCOUNT = 0.001
EPS = 1e-9
N_SEQS = 512
L = 256
PB = 64
NB = L // PB
TIA = PB * A
INV_LN2 = 1.4426950408889634
LOG2A = 4.392317422778761


def _mi_tiles_kernel(xi_ref, xj_ref, s1_ref, cnt_ref):
    xi = xi_ref[0]
    xj = xj_ref[0]
    counts = jax.lax.dot_general(
        xi, xj, (((0,), (0,)), ((), ())), preferred_element_type=jnp.float32)
    y = counts * (1.0 / N_SEQS) + EPS
    t = y * (jnp.log(y) * INV_LN2)
    acc = t[:, 0:128]
    for c in range(1, 10):
        acc = acc + t[:, 128 * c:128 * (c + 1)]
    part = acc[:, 0:PB] + acc[:, PB:128] + t[:, 20 * PB:TIA]
    s = part[0:PB, :]
    for a in range(1, A):
        s = s + part[PB * a:PB * (a + 1), :]
    s1_ref[0, 0] = s
    p = pl.program_id(0)
    i = ((p >= 4).astype(jnp.int32) + (p >= 7).astype(jnp.int32)
         + (p >= 9).astype(jnp.int32))
    j = p - (i * (9 - i)) // 2 + i

    @pl.when(i == j)
    def _():
        cnt_ref[...] = jnp.sum(xi.astype(jnp.float32), axis=0).reshape(1, 1, TIA)


def _finalize_kernel(cnt_ref, s1_ref, pssm_ref, cons_ref, coev_ref):
    cnt = cnt_ref[...]
    mean = cnt * (1.0 / N_SEQS)
    freq = mean + PSEUDOCOUNT
    fn = freq / jnp.sum(freq, axis=1, keepdims=True)
    pssm_ref[...] = jnp.log(fn * float(A)) * INV_LN2
    fe = mean + EPS
    neg_ent = jnp.sum(fe * (jnp.log(fe) * INV_LN2), axis=1, keepdims=True)
    cons_ref[...] = 1.0 + neg_ent * (1.0 / LOG2A)
    logp = jnp.log(fe) * INV_LN2
    c = jnp.sum((mean + A * EPS) * logp, axis=1, keepdims=True)
    ii = jax.lax.broadcasted_iota(jnp.int32, (L, L), 0)
    jj = jax.lax.broadcasted_iota(jnp.int32, (L, L), 1)
    bi = ii // PB
    bj = jj // PB
    u = s1_ref[...]
    up = jnp.where(bi <= bj, u, 0.0)
    s1 = up + jnp.transpose(jnp.where(bi < bj, up, 0.0))
    mi = s1 - c - jnp.transpose(c)
    ii = jax.lax.broadcasted_iota(jnp.int32, (L, L), 0)
    jj = jax.lax.broadcasted_iota(jnp.int32, (L, L), 1)
    mi = jnp.where(ii == jj, 0.0, mi)
    row_mean = jnp.mean(mi, axis=1, keepdims=True)
    col_mean = jnp.mean(mi, axis=0, keepdims=True)
    total = jnp.mean(mi)
    coev_ref[...] = mi - row_mean * col_mean / (total + EPS)


def _tri(p):
    i = ((p >= 4).astype(jnp.int32) + (p >= 7).astype(jnp.int32)
         + (p >= 9).astype(jnp.int32))
    j = p - (i * (9 - i)) // 2 + i
    return i, j


@jax.jit
def kernel(msa):
    mb = (msa.astype(jnp.bfloat16)
          .reshape(N_SEQS, NB, PB, A)
          .transpose(1, 0, 3, 2)
          .reshape(NB, N_SEQS, TIA))
    s1, cnt = pl.pallas_call(
        _mi_tiles_kernel,
        grid=(NB * (NB + 1) // 2,),
        in_specs=[
            pl.BlockSpec((1, N_SEQS, TIA), lambda p: (_tri(p)[0], 0, 0)),
            pl.BlockSpec((1, N_SEQS, TIA), lambda p: (_tri(p)[1], 0, 0)),
        ],
        out_specs=[
            pl.BlockSpec((1, 1, PB, PB), lambda p: (*_tri(p), 0, 0)),
            pl.BlockSpec((1, 1, TIA), lambda p: (_tri(p)[0], 0, 0)),
        ],
        out_shape=[
            jax.ShapeDtypeStruct((NB, NB, PB, PB), jnp.float32),
            jax.ShapeDtypeStruct((NB, 1, TIA), jnp.float32),
        ],
        compiler_params=pltpu.CompilerParams(
            dimension_semantics=("arbitrary",),
            vmem_limit_bytes=56 * 1024 * 1024,
        ),
        name="mi_tiles",
    )(mb, mb)
    s1 = s1.transpose(0, 2, 1, 3).reshape(L, L)
    cnt = cnt.reshape(NB, A, PB).transpose(0, 2, 1).reshape(L, A)
    pssm, cons, coev = pl.pallas_call(
        _finalize_kernel,
        out_shape=[
            jax.ShapeDtypeStruct((L, A), jnp.float32),
            jax.ShapeDtypeStruct((L, 1), jnp.float32),
            jax.ShapeDtypeStruct((L, L), jnp.float32),
        ],
        name="finalize",
    )(cnt, s1)
    return pssm, cons.reshape(L), coev
